# initial kernel scaffold (unmeasured)
import jax
import jax.numpy as jnp
from jax import lax
from jax.experimental import pallas as pl
from jax.experimental.pallas import tpu as pltpu


def kernel(
    x,
):
    def body(*refs):
        pass

    out_shape = jax.ShapeDtypeStruct(..., jnp.float32)
    return pl.pallas_call(body, out_shape=out_shape)(...)


# baseline (device time: 810490 ns/iter reference)
import jax
import jax.numpy as jnp
from jax import lax
from jax.experimental import pallas as pl
from jax.experimental.pallas import tpu as pltpu

M, N = 16384, 1024
HALF = M // 2
CHUNK = 2048
N_CHUNKS = HALF // CHUNK


def kernel(x):
    def body(x_ref, out_ref, a_ref, b_ref,
             p1_send, p1_recv, p2_send, p2_recv, copy_sems):
        xi = lax.axis_index("x")
        yi = lax.axis_index("y")
        zi = lax.axis_index("z")
        h = (xi + yi) % 2
        off_mine = h * HALF
        off_other = (1 - h) * HALF
        y_nbr = (xi, 1 - yi, zi)
        x_nbr = (1 - xi, yi, zi)

        barrier = pltpu.get_barrier_semaphore()
        for nbr in (y_nbr, x_nbr):
            pl.semaphore_signal(barrier, inc=1, device_id=nbr,
                                device_id_type=pl.DeviceIdType.MESH)
        pl.semaphore_wait(barrier, 2)

        rdma1 = pltpu.make_async_remote_copy(
            src_ref=x_ref.at[pl.ds(off_other, HALF), :],
            dst_ref=out_ref.at[pl.ds(off_other, HALF), :],
            send_sem=p1_send, recv_sem=p1_recv,
            device_id=y_nbr, device_id_type=pl.DeviceIdType.MESH,
        )
        rdma1.start()
        rdma1.wait()

        for c in range(N_CHUNKS):
            row = off_mine + c * CHUNK
            cp_a = pltpu.make_async_copy(
                out_ref.at[pl.ds(row, CHUNK), :], a_ref, copy_sems.at[0])
            cp_b = pltpu.make_async_copy(
                x_ref.at[pl.ds(row, CHUNK), :], b_ref, copy_sems.at[1])
            cp_a.start()
            cp_b.start()
            cp_a.wait()
            cp_b.wait()
            a_ref[...] = a_ref[...] + b_ref[...]
            cp_o = pltpu.make_async_copy(
                a_ref, out_ref.at[pl.ds(row, CHUNK), :], copy_sems.at[2])
            cp_o.start()
            cp_o.wait()

        rdma2 = pltpu.make_async_remote_copy(
            src_ref=out_ref.at[pl.ds(off_mine, HALF), :],
            dst_ref=out_ref.at[pl.ds(off_mine, HALF), :],
            send_sem=p2_send, recv_sem=p2_recv,
            device_id=x_nbr, device_id_type=pl.DeviceIdType.MESH,
        )
        rdma2.start()
        rdma2.wait()

    return pl.pallas_call(
        body,
        out_shape=jax.ShapeDtypeStruct((M, N), jnp.float32),
        in_specs=[pl.BlockSpec(memory_space=pl.ANY)],
        out_specs=pl.BlockSpec(memory_space=pl.ANY),
        scratch_shapes=[
            pltpu.VMEM((CHUNK, N), jnp.float32),
            pltpu.VMEM((CHUNK, N), jnp.float32),
            pltpu.SemaphoreType.DMA,
            pltpu.SemaphoreType.DMA,
            pltpu.SemaphoreType.DMA,
            pltpu.SemaphoreType.DMA,
            pltpu.SemaphoreType.DMA((3,)),
        ],
        compiler_params=pltpu.CompilerParams(collective_id=0),
    )(x)


# device time: 438591 ns/iter; 1.8479x vs baseline; 1.8479x over previous
import jax
import jax.numpy as jnp
from jax import lax
from jax.experimental import pallas as pl
from jax.experimental.pallas import tpu as pltpu

M, N = 16384, 1024
HALF = M // 2
N_CHUNKS = 16
CHUNK = HALF // N_CHUNKS


def kernel(x):
    def body(x_ref, out_ref, a_ref, b_ref,
             p1_send, p1_recv, p2_send, p2_recv, copy_sems):
        xi = lax.axis_index("x")
        yi = lax.axis_index("y")
        zi = lax.axis_index("z")
        h = (xi + yi) % 2
        off_mine = h * HALF
        off_other = (1 - h) * HALF
        y_nbr = (xi, 1 - yi, zi)
        x_nbr = (1 - xi, yi, zi)

        barrier = pltpu.get_barrier_semaphore()
        for nbr in (y_nbr, x_nbr):
            pl.semaphore_signal(barrier, inc=1, device_id=nbr,
                                device_id_type=pl.DeviceIdType.MESH)
        pl.semaphore_wait(barrier, 2)

        p1 = []
        for c in range(N_CHUNKS):
            row = off_other + c * CHUNK
            rdma = pltpu.make_async_remote_copy(
                src_ref=x_ref.at[pl.ds(row, CHUNK), :],
                dst_ref=out_ref.at[pl.ds(row, CHUNK), :],
                send_sem=p1_send.at[c], recv_sem=p1_recv.at[c],
                device_id=y_nbr, device_id_type=pl.DeviceIdType.MESH,
            )
            rdma.start()
            p1.append(rdma)

        p2 = []
        for c in range(N_CHUNKS):
            row = off_mine + c * CHUNK
            p1[c].wait_recv()
            cp_a = pltpu.make_async_copy(
                out_ref.at[pl.ds(row, CHUNK), :], a_ref, copy_sems.at[0])
            cp_b = pltpu.make_async_copy(
                x_ref.at[pl.ds(row, CHUNK), :], b_ref, copy_sems.at[1])
            cp_a.start()
            cp_b.start()
            cp_a.wait()
            cp_b.wait()
            a_ref[...] = a_ref[...] + b_ref[...]
            cp_o = pltpu.make_async_copy(
                a_ref, out_ref.at[pl.ds(row, CHUNK), :], copy_sems.at[2])
            cp_o.start()
            cp_o.wait()
            rdma = pltpu.make_async_remote_copy(
                src_ref=out_ref.at[pl.ds(row, CHUNK), :],
                dst_ref=out_ref.at[pl.ds(row, CHUNK), :],
                send_sem=p2_send.at[c], recv_sem=p2_recv.at[c],
                device_id=x_nbr, device_id_type=pl.DeviceIdType.MESH,
            )
            rdma.start()
            p2.append(rdma)

        for c in range(N_CHUNKS):
            p1[c].wait_send()
            p2[c].wait()

    return pl.pallas_call(
        body,
        out_shape=jax.ShapeDtypeStruct((M, N), jnp.float32),
        in_specs=[pl.BlockSpec(memory_space=pl.ANY)],
        out_specs=pl.BlockSpec(memory_space=pl.ANY),
        scratch_shapes=[
            pltpu.VMEM((CHUNK, N), jnp.float32),
            pltpu.VMEM((CHUNK, N), jnp.float32),
            pltpu.SemaphoreType.DMA((N_CHUNKS,)),
            pltpu.SemaphoreType.DMA((N_CHUNKS,)),
            pltpu.SemaphoreType.DMA((N_CHUNKS,)),
            pltpu.SemaphoreType.DMA((N_CHUNKS,)),
            pltpu.SemaphoreType.DMA((3,)),
        ],
        compiler_params=pltpu.CompilerParams(collective_id=0),
    )(x)


# device time: 426760 ns/iter; 1.8992x vs baseline; 1.0277x over previous
import jax
import jax.numpy as jnp
from jax import lax
from jax.experimental import pallas as pl
from jax.experimental.pallas import tpu as pltpu

M, N = 16384, 1024
HALF = M // 2
N_CHUNKS = 32
CHUNK = HALF // N_CHUNKS


def kernel(x):
    def body(x_ref, out_ref, a_ref, b_ref,
             p1_send, p1_recv, p2_send, p2_recv, copy_sems):
        xi = lax.axis_index("x")
        yi = lax.axis_index("y")
        zi = lax.axis_index("z")
        h = (xi + yi) % 2
        off_mine = h * HALF
        off_other = (1 - h) * HALF
        y_nbr = (xi, 1 - yi, zi)
        x_nbr = (1 - xi, yi, zi)

        barrier = pltpu.get_barrier_semaphore()
        for nbr in (y_nbr, x_nbr):
            pl.semaphore_signal(barrier, inc=1, device_id=nbr,
                                device_id_type=pl.DeviceIdType.MESH)
        pl.semaphore_wait(barrier, 2)

        p1 = []
        for c in range(N_CHUNKS):
            row = off_other + c * CHUNK
            rdma = pltpu.make_async_remote_copy(
                src_ref=x_ref.at[pl.ds(row, CHUNK), :],
                dst_ref=out_ref.at[pl.ds(row, CHUNK), :],
                send_sem=p1_send.at[c], recv_sem=p1_recv.at[c],
                device_id=y_nbr, device_id_type=pl.DeviceIdType.MESH,
            )
            rdma.start()
            p1.append(rdma)

        p2 = []
        for c in range(N_CHUNKS):
            row = off_mine + c * CHUNK
            p1[c].wait_recv()
            cp_a = pltpu.make_async_copy(
                out_ref.at[pl.ds(row, CHUNK), :], a_ref, copy_sems.at[0])
            cp_b = pltpu.make_async_copy(
                x_ref.at[pl.ds(row, CHUNK), :], b_ref, copy_sems.at[1])
            cp_a.start()
            cp_b.start()
            cp_a.wait()
            cp_b.wait()
            a_ref[...] = a_ref[...] + b_ref[...]
            cp_o = pltpu.make_async_copy(
                a_ref, out_ref.at[pl.ds(row, CHUNK), :], copy_sems.at[2])
            cp_o.start()
            cp_o.wait()
            rdma = pltpu.make_async_remote_copy(
                src_ref=out_ref.at[pl.ds(row, CHUNK), :],
                dst_ref=out_ref.at[pl.ds(row, CHUNK), :],
                send_sem=p2_send.at[c], recv_sem=p2_recv.at[c],
                device_id=x_nbr, device_id_type=pl.DeviceIdType.MESH,
            )
            rdma.start()
            p2.append(rdma)

        for c in range(N_CHUNKS):
            p1[c].wait_send()
            p2[c].wait()

    return pl.pallas_call(
        body,
        out_shape=jax.ShapeDtypeStruct((M, N), jnp.float32),
        in_specs=[pl.BlockSpec(memory_space=pl.ANY)],
        out_specs=pl.BlockSpec(memory_space=pl.ANY),
        scratch_shapes=[
            pltpu.VMEM((CHUNK, N), jnp.float32),
            pltpu.VMEM((CHUNK, N), jnp.float32),
            pltpu.SemaphoreType.DMA((N_CHUNKS,)),
            pltpu.SemaphoreType.DMA((N_CHUNKS,)),
            pltpu.SemaphoreType.DMA((N_CHUNKS,)),
            pltpu.SemaphoreType.DMA((N_CHUNKS,)),
            pltpu.SemaphoreType.DMA((3,)),
        ],
        compiler_params=pltpu.CompilerParams(collective_id=0),
    )(x)


# device time: 365216 ns/iter; 2.2192x vs baseline; 1.1685x over previous
import jax
import jax.numpy as jnp
from jax import lax
from jax.experimental import pallas as pl
from jax.experimental.pallas import tpu as pltpu

M, N = 16384, 1024
QROWS = M // 4
K = 8
CH = QROWS // K
MESH = pl.DeviceIdType.MESH


def kernel(x):
    def body(x_ref, out_ref, rem_ref, a_ref, b_ref,
             y_snd, y_rcv, x_snd, x_rcv, z_snd, z_rcv, cp_sems):
        xi = lax.axis_index("x")
        yi = lax.axis_index("y")
        zi = lax.axis_index("z")
        zb = zi % 2
        zn = zi + 1 - 2 * zb
        B = (xi, 1 - yi, zi)
        C = (1 - xi, yi, zi)
        E = (xi, yi, zn)
        qrow = (2 * xi + zb) * QROWS
        qx = (2 * (1 - xi) + zb) * QROWS
        qz = (2 * xi + (1 - zb)) * QROWS
        qd = (2 * (1 - xi) + (1 - zb)) * QROWS

        barrier = pltpu.get_barrier_semaphore()
        for nbr in (B, C, E):
            pl.semaphore_signal(barrier, inc=1, device_id=nbr,
                                device_id_type=MESH)
        pl.semaphore_wait(barrier, 3)

        def rcopy(rows, ssem, rsem, dev, from_x=False):
            src = (x_ref if from_x else rem_ref).at[pl.ds(rows, CH), :]
            return pltpu.make_async_remote_copy(
                src_ref=src, dst_ref=rem_ref.at[pl.ds(rows, CH), :],
                send_sem=ssem, recv_sem=rsem, device_id=dev,
                device_id_type=MESH)

        def add(rows):
            ca = pltpu.make_async_copy(
                x_ref.at[pl.ds(rows, CH), :], a_ref, cp_sems.at[0])
            cb = pltpu.make_async_copy(
                rem_ref.at[pl.ds(rows, CH), :], b_ref, cp_sems.at[1])
            ca.start()
            cb.start()
            ca.wait()
            cb.wait()
            a_ref[...] = a_ref[...] + b_ref[...]
            co = pltpu.make_async_copy(
                a_ref, out_ref.at[pl.ds(rows, CH), :], cp_sems.at[2])
            co.start()
            co.wait()

        y_rd = []
        for k in range(K):
            r = rcopy(qrow + k * CH, y_snd.at[k], y_rcv.at[k], B,
                      from_x=True)
            r.start()
            y_rd.append(r)

        x_rd = [None] * (K + K // 2)
        z_rd = [None] * (K + K // 2)
        for k in range(K):
            y_rd[k].wait_recv()
            rx = rcopy(qrow + k * CH, x_snd.at[k], x_rcv.at[k], C)
            rx.start()
            x_rd[k] = rx
            rz = rcopy(qrow + k * CH, z_snd.at[k], z_rcv.at[k], E)
            rz.start()
            z_rd[k] = rz
            add(qrow + k * CH)

        for k in range(K):
            if k < K // 2:
                z_rd[k].wait_recv()
                rf = rcopy(qz + k * CH, x_snd.at[K + k], x_rcv.at[K + k], C)
                rf.start()
                x_rd[K + k] = rf
                add(qz + k * CH)
            else:
                x_rd[k].wait_recv()
                j = k - K // 2
                rf = rcopy(qx + k * CH, z_snd.at[K + j], z_rcv.at[K + j], E)
                rf.start()
                z_rd[K + j] = rf
                add(qx + k * CH)

        for k in range(K // 2):
            x_rd[k].wait_recv()
            add(qx + k * CH)
        for k in range(K // 2, K):
            z_rd[k].wait_recv()
            add(qz + k * CH)
        for k in range(K):
            if k < K // 2:
                x_rd[K + k].wait_recv()
            else:
                z_rd[K + k - K // 2].wait_recv()
            add(qd + k * CH)

        for r in y_rd + x_rd + z_rd:
            r.wait_send()

    out, _rem = pl.pallas_call(
        body,
        out_shape=(
            jax.ShapeDtypeStruct((M, N), jnp.float32),
            jax.ShapeDtypeStruct((M, N), jnp.float32),
        ),
        in_specs=[pl.BlockSpec(memory_space=pl.ANY)],
        out_specs=(
            pl.BlockSpec(memory_space=pl.ANY),
            pl.BlockSpec(memory_space=pl.ANY),
        ),
        scratch_shapes=[
            pltpu.VMEM((CH, N), jnp.float32),
            pltpu.VMEM((CH, N), jnp.float32),
            pltpu.SemaphoreType.DMA((K,)),
            pltpu.SemaphoreType.DMA((K,)),
            pltpu.SemaphoreType.DMA((K + K // 2,)),
            pltpu.SemaphoreType.DMA((K + K // 2,)),
            pltpu.SemaphoreType.DMA((K + K // 2,)),
            pltpu.SemaphoreType.DMA((K + K // 2,)),
            pltpu.SemaphoreType.DMA((3,)),
        ],
        compiler_params=pltpu.CompilerParams(collective_id=0),
    )(x)
    return out


# device time: 360666 ns/iter; 2.2472x vs baseline; 1.0126x over previous
import jax
import jax.numpy as jnp
from jax import lax
from jax.experimental import pallas as pl
from jax.experimental.pallas import tpu as pltpu

M, N = 16384, 1024
QROWS = M // 4
K = 8
CH = QROWS // K
MESH = pl.DeviceIdType.MESH


def kernel(x):
    def body(x_ref, out_ref, rem_ref, a_ref, b_ref,
             y_snd, y_rcv, x_snd, x_rcv, z_snd, z_rcv, ld_sems, st_sems):
        xi = lax.axis_index("x")
        yi = lax.axis_index("y")
        zi = lax.axis_index("z")
        zb = zi % 2
        zn = zi + 1 - 2 * zb
        B = (xi, 1 - yi, zi)
        C = (1 - xi, yi, zi)
        E = (xi, yi, zn)
        qrow = (2 * xi + zb) * QROWS
        qx = (2 * (1 - xi) + zb) * QROWS
        qz = (2 * xi + (1 - zb)) * QROWS
        qd = (2 * (1 - xi) + (1 - zb)) * QROWS

        barrier = pltpu.get_barrier_semaphore()
        for nbr in (B, C, E):
            pl.semaphore_signal(barrier, inc=1, device_id=nbr,
                                device_id_type=MESH)
        pl.semaphore_wait(barrier, 3)

        def rcopy(rows, ssem, rsem, dev, from_x=False):
            src = (x_ref if from_x else rem_ref).at[pl.ds(rows, CH), :]
            return pltpu.make_async_remote_copy(
                src_ref=src, dst_ref=rem_ref.at[pl.ds(rows, CH), :],
                send_sem=ssem, recv_sem=rsem, device_id=dev,
                device_id_type=MESH)

        pipe = {"i": 0, "loads": [None, None], "rows": [None, None],
                "stores": [None, None]}

        def _finish(slot):
            la, lb = pipe["loads"][slot]
            la.wait()
            lb.wait()
            a_ref[slot] = a_ref[slot] + b_ref[slot]
            co = pltpu.make_async_copy(
                a_ref.at[slot],
                out_ref.at[pl.ds(pipe["rows"][slot], CH), :],
                st_sems.at[slot])
            co.start()
            pipe["stores"][slot] = co
            pipe["loads"][slot] = None

        def add(rows):
            s = pipe["i"] % 2
            if pipe["stores"][s] is not None:
                pipe["stores"][s].wait()
                pipe["stores"][s] = None
            ca = pltpu.make_async_copy(
                x_ref.at[pl.ds(rows, CH), :], a_ref.at[s], ld_sems.at[s])
            cb = pltpu.make_async_copy(
                rem_ref.at[pl.ds(rows, CH), :], b_ref.at[s],
                ld_sems.at[2 + s])
            ca.start()
            cb.start()
            pipe["loads"][s] = (ca, cb)
            pipe["rows"][s] = rows
            prev = (pipe["i"] - 1) % 2
            if pipe["i"] >= 1 and pipe["loads"][prev] is not None:
                _finish(prev)
            pipe["i"] += 1

        def add_flush():
            last = (pipe["i"] - 1) % 2
            if pipe["loads"][last] is not None:
                _finish(last)
            for s in (0, 1):
                if pipe["stores"][s] is not None:
                    pipe["stores"][s].wait()
                    pipe["stores"][s] = None

        y_rd = []
        for k in range(K):
            r = rcopy(qrow + k * CH, y_snd.at[k], y_rcv.at[k], B,
                      from_x=True)
            r.start()
            y_rd.append(r)

        x_rd = [None] * (K + K // 2)
        z_rd = [None] * (K + K // 2)
        for k in range(K):
            y_rd[k].wait_recv()
            rx = rcopy(qrow + k * CH, x_snd.at[k], x_rcv.at[k], C)
            rx.start()
            x_rd[k] = rx
            rz = rcopy(qrow + k * CH, z_snd.at[k], z_rcv.at[k], E)
            rz.start()
            z_rd[k] = rz
            add(qrow + k * CH)

        for k in range(K):
            if k < K // 2:
                z_rd[k].wait_recv()
                rf = rcopy(qz + k * CH, x_snd.at[K + k], x_rcv.at[K + k], C)
                rf.start()
                x_rd[K + k] = rf
                add(qz + k * CH)
            else:
                x_rd[k].wait_recv()
                j = k - K // 2
                rf = rcopy(qx + k * CH, z_snd.at[K + j], z_rcv.at[K + j], E)
                rf.start()
                z_rd[K + j] = rf
                add(qx + k * CH)

        for k in range(K // 2):
            x_rd[k].wait_recv()
            add(qx + k * CH)
        for k in range(K // 2, K):
            z_rd[k].wait_recv()
            add(qz + k * CH)
        for k in range(K):
            if k < K // 2:
                x_rd[K + k].wait_recv()
            else:
                z_rd[K + k - K // 2].wait_recv()
            add(qd + k * CH)

        add_flush()
        for r in y_rd + x_rd + z_rd:
            r.wait_send()

    out, _rem = pl.pallas_call(
        body,
        out_shape=(
            jax.ShapeDtypeStruct((M, N), jnp.float32),
            jax.ShapeDtypeStruct((M, N), jnp.float32),
        ),
        in_specs=[pl.BlockSpec(memory_space=pl.ANY)],
        out_specs=(
            pl.BlockSpec(memory_space=pl.ANY),
            pl.BlockSpec(memory_space=pl.ANY),
        ),
        scratch_shapes=[
            pltpu.VMEM((2, CH, N), jnp.float32),
            pltpu.VMEM((2, CH, N), jnp.float32),
            pltpu.SemaphoreType.DMA((K,)),
            pltpu.SemaphoreType.DMA((K,)),
            pltpu.SemaphoreType.DMA((K + K // 2,)),
            pltpu.SemaphoreType.DMA((K + K // 2,)),
            pltpu.SemaphoreType.DMA((K + K // 2,)),
            pltpu.SemaphoreType.DMA((K + K // 2,)),
            pltpu.SemaphoreType.DMA((4,)),
            pltpu.SemaphoreType.DMA((2,)),
        ],
        compiler_params=pltpu.CompilerParams(collective_id=0),
    )(x)
    return out


# device time: 317893 ns/iter; 2.5496x vs baseline; 1.1346x over previous
import jax
import jax.numpy as jnp
from jax import lax
from jax.experimental import pallas as pl
from jax.experimental.pallas import tpu as pltpu

M, N = 16384, 1024
QROWS = M // 4
K = 16
CH = QROWS // K
DY = 6
FX = 5
FZ = K - DY - FX
MESH = pl.DeviceIdType.MESH


def kernel(x):
    def body(x_ref, out_ref, rem_ref, a_ref, b_ref,
             y_snd, y_rcv, x_snd, x_rcv, z_snd, z_rcv, ld_sems, st_sems):
        xi = lax.axis_index("x")
        yi = lax.axis_index("y")
        zi = lax.axis_index("z")
        zb = zi % 2
        zn = zi + 1 - 2 * zb
        B = (xi, 1 - yi, zi)
        C = (1 - xi, yi, zi)
        E = (xi, yi, zn)
        qrow = (2 * xi + zb) * QROWS
        qx = (2 * (1 - xi) + zb) * QROWS
        qz = (2 * xi + (1 - zb)) * QROWS
        qd = (2 * (1 - xi) + (1 - zb)) * QROWS

        barrier = pltpu.get_barrier_semaphore()
        for nbr in (B, C, E):
            pl.semaphore_signal(barrier, inc=1, device_id=nbr,
                                device_id_type=MESH)
        pl.semaphore_wait(barrier, 3)

        def rcopy(rows, ssem, rsem, dev, from_x=False):
            src = (x_ref if from_x else rem_ref).at[pl.ds(rows, CH), :]
            return pltpu.make_async_remote_copy(
                src_ref=src, dst_ref=rem_ref.at[pl.ds(rows, CH), :],
                send_sem=ssem, recv_sem=rsem, device_id=dev,
                device_id_type=MESH)

        pipe = {"i": 0, "loads": [None, None], "rows": [None, None],
                "stores": [None, None]}

        def _finish(slot):
            la, lb = pipe["loads"][slot]
            la.wait()
            lb.wait()
            a_ref[slot] = a_ref[slot] + b_ref[slot]
            co = pltpu.make_async_copy(
                a_ref.at[slot],
                out_ref.at[pl.ds(pipe["rows"][slot], CH), :],
                st_sems.at[slot])
            co.start()
            pipe["stores"][slot] = co
            pipe["loads"][slot] = None

        def add(rows):
            s = pipe["i"] % 2
            if pipe["stores"][s] is not None:
                pipe["stores"][s].wait()
                pipe["stores"][s] = None
            ca = pltpu.make_async_copy(
                x_ref.at[pl.ds(rows, CH), :], a_ref.at[s], ld_sems.at[s])
            cb = pltpu.make_async_copy(
                rem_ref.at[pl.ds(rows, CH), :], b_ref.at[s],
                ld_sems.at[2 + s])
            ca.start()
            cb.start()
            pipe["loads"][s] = (ca, cb)
            pipe["rows"][s] = rows
            prev = (pipe["i"] - 1) % 2
            if pipe["i"] >= 1 and pipe["loads"][prev] is not None:
                _finish(prev)
            pipe["i"] += 1

        def add_flush():
            last = (pipe["i"] - 1) % 2
            if pipe["loads"][last] is not None:
                _finish(last)
            for s in (0, 1):
                if pipe["stores"][s] is not None:
                    pipe["stores"][s].wait()
                    pipe["stores"][s] = None

        y_rd = []
        for k in range(K):
            r = rcopy(qrow + k * CH, y_snd.at[k], y_rcv.at[k], B,
                      from_x=True)
            r.start()
            y_rd.append(r)
        for i in range(DY):
            r = rcopy(qd + i * CH, y_snd.at[K + i], y_rcv.at[K + i], B,
                      from_x=True)
            r.start()
            y_rd.append(r)

        x_rd = [None] * (K + FX)
        z_rd = [None] * (K + FZ)
        for k in range(K):
            y_rd[k].wait_recv()
            rx = rcopy(qrow + k * CH, x_snd.at[k], x_rcv.at[k], C)
            rx.start()
            x_rd[k] = rx
            rz = rcopy(qrow + k * CH, z_snd.at[k], z_rcv.at[k], E)
            rz.start()
            z_rd[k] = rz
            add(qrow + k * CH)

        for j in range(DY, K):
            if j < DY + FX:
                z_rd[j].wait_recv()
                i = j - DY
                rf = rcopy(qz + j * CH, x_snd.at[K + i], x_rcv.at[K + i], C)
                rf.start()
                x_rd[K + i] = rf
                add(qz + j * CH)
            else:
                x_rd[j].wait_recv()
                i = j - DY - FX
                rf = rcopy(qx + j * CH, z_snd.at[K + i], z_rcv.at[K + i], E)
                rf.start()
                z_rd[K + i] = rf
                add(qx + j * CH)

        for k in range(DY + FX):
            x_rd[k].wait_recv()
            add(qx + k * CH)
        for k in range(K):
            if not (DY <= k < DY + FX):
                z_rd[k].wait_recv()
                add(qz + k * CH)
        for k in range(K):
            if k < DY:
                y_rd[K + k].wait_recv()
            elif k < DY + FX:
                x_rd[K + k - DY].wait_recv()
            else:
                z_rd[K + k - DY - FX].wait_recv()
            add(qd + k * CH)

        add_flush()
        for r in y_rd + x_rd + z_rd:
            r.wait_send()

    out, _rem = pl.pallas_call(
        body,
        out_shape=(
            jax.ShapeDtypeStruct((M, N), jnp.float32),
            jax.ShapeDtypeStruct((M, N), jnp.float32),
        ),
        in_specs=[pl.BlockSpec(memory_space=pl.ANY)],
        out_specs=(
            pl.BlockSpec(memory_space=pl.ANY),
            pl.BlockSpec(memory_space=pl.ANY),
        ),
        scratch_shapes=[
            pltpu.VMEM((2, CH, N), jnp.float32),
            pltpu.VMEM((2, CH, N), jnp.float32),
            pltpu.SemaphoreType.DMA((K + DY,)),
            pltpu.SemaphoreType.DMA((K + DY,)),
            pltpu.SemaphoreType.DMA((K + FX,)),
            pltpu.SemaphoreType.DMA((K + FX,)),
            pltpu.SemaphoreType.DMA((K + FZ,)),
            pltpu.SemaphoreType.DMA((K + FZ,)),
            pltpu.SemaphoreType.DMA((4,)),
            pltpu.SemaphoreType.DMA((2,)),
        ],
        compiler_params=pltpu.CompilerParams(collective_id=0),
    )(x)
    return out
